# baseline (speedup 1.0000x reference)
import functools

import jax
import jax.numpy as jnp
from jax import lax
from jax.experimental import pallas as pl
from jax.experimental.pallas import tpu as pltpu
from jax.experimental.pallas import tpu_sc as plsc

K = 8
BIG = 1e9
N_BLK = 512
LANES = 128
T_TILE = 256
QT = 16
NCH = 12
BCAP = 1024


def _insert_stream(md, base_col, vals, inds, rows):
    cur_v = [vals[s] for s in range(K)]
    cur_i = [inds[s] for s in range(K)]
    for g in range(N_BLK // LANES):
        v = md[:, g * LANES:(g + 1) * LANES]
        vidx = (base_col + g * LANES
                + lax.broadcasted_iota(jnp.int32, (rows, LANES), 1))
        c = [v < cur_v[s] for s in range(K)]
        new_v, new_i = [], []
        for s in range(K):
            if s == 0:
                new_v.append(jnp.where(c[0], v, cur_v[0]))
                new_i.append(jnp.where(c[0], vidx, cur_i[0]))
            else:
                new_v.append(jnp.where(c[s], jnp.where(c[s - 1], cur_v[s - 1],
                                                       v), cur_v[s]))
                new_i.append(jnp.where(c[s], jnp.where(c[s - 1], cur_i[s - 1],
                                                       vidx), cur_i[s]))
        cur_v, cur_i = new_v, new_i
    for s in range(K):
        vals[s] = cur_v[s]
        inds[s] = cur_i[s]


def _masked_dist(q2, p2, code, ph, hb, pb):
    cross = lax.dot_general(
        hb.astype(jnp.bfloat16), pb.astype(jnp.bfloat16),
        dimension_numbers=(((1,), (1,)), ((), ())),
        preferred_element_type=jnp.float32,
    )
    d2 = (q2 + p2) - 2.0 * cross
    dist = jnp.sqrt(jnp.maximum(d2, 1e-12))
    return jnp.where(code == ph, dist, jnp.float32(BIG))


def _merge_and_softmax(vals, inds, idx_out_ref, w_out_ref):
    cv = jnp.stack([vals[s] for s in range(K)])
    ci = jnp.stack([inds[s] for s in range(K)])
    outd, outi = [], []
    for _ in range(K):
        m = jnp.min(jnp.min(cv, axis=0), axis=1, keepdims=True)
        eq = cv == m[None, :, :]
        imin = jnp.min(jnp.min(jnp.where(eq, ci, jnp.int32(2**30)), axis=0),
                       axis=1, keepdims=True)
        outd.append(m)
        outi.append(imin)
        kill = eq & (ci == imin[None, :, :])
        cv = jnp.where(kill, jnp.inf, cv)
    topd = jnp.concatenate(outd, axis=1)
    topi = jnp.concatenate(outi, axis=1)
    unnorm = jnp.exp(-(topd - topd[:, 0:1]))
    w = unnorm / jnp.sum(unnorm, axis=1, keepdims=True)
    idx_out_ref[...] = topi
    w_out_ref[...] = w


def _dense_body(q2_ref, ph_ref, p2_ref, code_ref, h_ref, pr_ref,
                idx_out_ref, w_out_ref, vals, inds, *, n_blocks):
    j = pl.program_id(1)

    @pl.when(j == 0)
    def _init():
        vals[...] = jnp.full((K, T_TILE, LANES), jnp.inf, jnp.float32)
        inds[...] = jnp.zeros((K, T_TILE, LANES), jnp.int32)

    md = _masked_dist(q2_ref[...], p2_ref[...], code_ref[...], ph_ref[...],
                      h_ref[...], pr_ref[...])
    _insert_stream(md, j * N_BLK, vals, inds, T_TILE)

    @pl.when(j == n_blocks - 1)
    def _merge():
        _merge_and_softmax(vals, inds, idx_out_ref, w_out_ref)


def _bucket_body(pb0_ref, nch_ref, q2_ref, ph_ref, p2_ref, code_ref, h_ref,
                 pr_ref, idx_out_ref, w_out_ref, vals, inds):
    i = pl.program_id(0)
    c = pl.program_id(1)

    @pl.when(c == 0)
    def _init():
        vals[...] = jnp.full((K, QT, LANES), jnp.inf, jnp.float32)
        inds[...] = jnp.zeros((K, QT, LANES), jnp.int32)

    @pl.when(c < nch_ref[i])
    def _insert():
        blk = pb0_ref[i] + jnp.minimum(c, nch_ref[i] - 1)
        md = _masked_dist(q2_ref[...], p2_ref[...], code_ref[...], ph_ref[...],
                          h_ref[...], pr_ref[...])
        _insert_stream(md, blk * N_BLK, vals, inds, QT)

    @pl.when(c == NCH - 1)
    def _merge():
        _merge_and_softmax(vals, inds, idx_out_ref, w_out_ref)


def _fast_blk(i, c, pb0, nch):
    return pb0[i] + jnp.minimum(c, nch[i] - 1)


def _run_bucket_topk(pb0, nch, q2_s, phones_s2d, p2_s2d, code_s2d, h_s,
                     p_sorted):
    T, D = h_s.shape
    grid_spec = pltpu.PrefetchScalarGridSpec(
        num_scalar_prefetch=2,
        grid=(T // QT, NCH),
        in_specs=[
            pl.BlockSpec((QT, 1), lambda i, c, pb0, nch: (i, 0)),
            pl.BlockSpec((QT, 1), lambda i, c, pb0, nch: (i, 0)),
            pl.BlockSpec((1, N_BLK),
                         lambda i, c, pb0, nch: (0, _fast_blk(i, c, pb0, nch))),
            pl.BlockSpec((1, N_BLK),
                         lambda i, c, pb0, nch: (0, _fast_blk(i, c, pb0, nch))),
            pl.BlockSpec((QT, D), lambda i, c, pb0, nch: (i, 0)),
            pl.BlockSpec((N_BLK, D),
                         lambda i, c, pb0, nch: (_fast_blk(i, c, pb0, nch), 0)),
        ],
        out_specs=[
            pl.BlockSpec((QT, K), lambda i, c, pb0, nch: (i, 0)),
            pl.BlockSpec((QT, K), lambda i, c, pb0, nch: (i, 0)),
        ],
        scratch_shapes=[
            pltpu.VMEM((K, QT, LANES), jnp.float32),
            pltpu.VMEM((K, QT, LANES), jnp.int32),
        ],
    )
    return pl.pallas_call(
        _bucket_body,
        grid_spec=grid_spec,
        out_shape=[
            jax.ShapeDtypeStruct((T, K), jnp.int32),
            jax.ShapeDtypeStruct((T, K), jnp.float32),
        ],
        compiler_params=pltpu.CompilerParams(
            dimension_semantics=("arbitrary", "arbitrary"),
        ),
    )(pb0, nch, q2_s, phones_s2d, p2_s2d, code_s2d, h_s, p_sorted)


def _run_dense_topk(q2, phones2d, p2, code2d, h_clean, prototypes):
    T, D = h_clean.shape
    N = prototypes.shape[0]
    n_t, n_blocks = T // T_TILE, N // N_BLK
    kern = functools.partial(_dense_body, n_blocks=n_blocks)
    return pl.pallas_call(
        kern,
        grid=(n_t, n_blocks),
        in_specs=[
            pl.BlockSpec((T_TILE, 1), lambda i, j: (i, 0)),
            pl.BlockSpec((T_TILE, 1), lambda i, j: (i, 0)),
            pl.BlockSpec((1, N_BLK), lambda i, j: (0, j)),
            pl.BlockSpec((1, N_BLK), lambda i, j: (0, j)),
            pl.BlockSpec((T_TILE, D), lambda i, j: (i, 0)),
            pl.BlockSpec((N_BLK, D), lambda i, j: (j, 0)),
        ],
        out_specs=[
            pl.BlockSpec((T_TILE, K), lambda i, j: (i, 0)),
            pl.BlockSpec((T_TILE, K), lambda i, j: (i, 0)),
        ],
        out_shape=[
            jax.ShapeDtypeStruct((T, K), jnp.int32),
            jax.ShapeDtypeStruct((T, K), jnp.float32),
        ],
        scratch_shapes=[
            pltpu.VMEM((K, T_TILE, LANES), jnp.float32),
            pltpu.VMEM((K, T_TILE, LANES), jnp.int32),
        ],
        compiler_params=pltpu.CompilerParams(
            dimension_semantics=("arbitrary", "arbitrary"),
        ),
    )(q2, phones2d, p2, code2d, h_clean, prototypes)


def _make_sc_gather(V, D, B, chunk=None):
    nw = 32
    b_per_w = B // nw
    chunk = chunk or b_per_w
    n_ch = b_per_w // chunk
    mesh = plsc.VectorSubcoreMesh(core_axis_name="c", subcore_axis_name="s")

    @functools.partial(
        pl.kernel, mesh=mesh,
        out_type=jax.ShapeDtypeStruct((B, D), jnp.float32),
        scratch_types=[
            pltpu.VMEM((chunk,), jnp.int32),
            pltpu.VMEM((chunk, D), jnp.float32),
            pltpu.SemaphoreType.DMA,
        ],
    )
    def sc_gather(table_hbm, idx_hbm, out_hbm, idx_v, rows_v, sem):
        wid = lax.axis_index("s") * 2 + lax.axis_index("c")
        for ch in range(n_ch):
            base = wid * b_per_w + ch * chunk
            pltpu.sync_copy(idx_hbm.at[pl.ds(base, chunk)], idx_v)
            pltpu.async_copy(table_hbm.at[idx_v], rows_v, sem).wait()
            pltpu.sync_copy(rows_v, out_hbm.at[pl.ds(base, chunk)])

    return sc_gather


def _make_sc_gather_seg(D, b_seg, seg):
    nw = 32
    b_per_w = b_seg // nw
    mesh = plsc.VectorSubcoreMesh(core_axis_name="c", subcore_axis_name="s")

    @functools.partial(
        pl.kernel, mesh=mesh,
        out_type=jax.ShapeDtypeStruct((b_seg, D), jnp.float32),
        scratch_types=[
            pltpu.VMEM((b_per_w,), jnp.int32),
            pltpu.VMEM((b_per_w, D), jnp.float32),
            pltpu.SemaphoreType.DMA,
        ],
    )
    def sc_gather_seg(table_hbm, idx_hbm, out_hbm, idx_v, rows_v, sem):
        wid = lax.axis_index("s") * 2 + lax.axis_index("c")
        pltpu.sync_copy(idx_hbm.at[pl.ds(seg * b_seg + wid * b_per_w, b_per_w)],
                        idx_v)
        pltpu.async_copy(table_hbm.at[idx_v], rows_v, sem).wait()
        pltpu.sync_copy(rows_v, out_hbm.at[pl.ds(wid * b_per_w, b_per_w)])

    return sc_gather_seg


def _combine_body(g_ref, w_ref, out_ref):
    g = g_ref[...]
    w = w_ref[...]
    out_ref[...] = jnp.sum(w[:, :, None] * g, axis=1)


def _run_combine(gathered, w):
    T = w.shape[0]
    g3 = gathered.reshape(T, K, 256)
    return pl.pallas_call(
        _combine_body,
        grid=(T // T_TILE,),
        in_specs=[
            pl.BlockSpec((T_TILE, K, 256), lambda i: (i, 0, 0)),
            pl.BlockSpec((T_TILE, K), lambda i: (i, 0)),
        ],
        out_specs=pl.BlockSpec((T_TILE, 256), lambda i: (i, 0)),
        out_shape=jax.ShapeDtypeStruct((T, 256), jnp.float32),
    )(g3, w)


def kernel(h_clean, phones, target_gender, prototypes, proto_phones,
           proto_genders):
    T, D = h_clean.shape
    N = prototypes.shape[0]
    phones_i = phones.astype(jnp.int32)

    code_v = jnp.where(proto_genders == target_gender, proto_phones,
                       jnp.int32(64)).astype(jnp.int32)
    code_blk = code_v.reshape(N // LANES, LANES)
    oh3 = (code_blk[:, :, None]
           == jnp.arange(64, dtype=jnp.int32)[None, None, :]).astype(jnp.int32)
    blk_cnt = oh3.sum(axis=1)
    counts64 = blk_cnt.sum(axis=0)

    qperm = jnp.argsort(phones_i, stable=True).astype(jnp.int32)
    phones_s = jnp.sort(phones_i)
    pfirst = phones_s[0::QT]
    plast = phones_s[QT - 1::QT]
    nch = (2 * (plast - pfirst + 1)).astype(jnp.int32)
    pb0 = (2 * pfirst).astype(jnp.int32)

    fast_ok = (jnp.max(counts64) <= BCAP) & (jnp.max(nch) <= NCH)

    def fast_path():
        blk_base = jnp.cumsum(blk_cnt, axis=0) - blk_cnt
        within = jnp.cumsum(oh3, axis=1) - oh3
        rank = (oh3 * (blk_base[:, None, :] + within)).sum(axis=2).reshape(N)
        pos = jnp.where((code_v < 64) & (rank < BCAP),
                        code_v * BCAP + rank, N)
        packed_src = jnp.arange(N, dtype=jnp.int32) | ((code_v + 2) << 16)
        packed = jnp.zeros(N, jnp.int32).at[pos].set(packed_src, mode="drop")
        perm_arr = packed & jnp.int32(0xFFFF)
        code_arr = (packed >> 16) - 2
        p_sorted = jnp.concatenate(
            [_make_sc_gather_seg(D, 8192, seg)(prototypes, perm_arr)
             for seg in range(N // 8192)], axis=0)
        p2_s = jnp.sum(p_sorted * p_sorted, axis=1)
        h_s = _make_sc_gather(T, D, T)(h_clean, qperm)
        q2_s = jnp.sum(h_s * h_s, axis=1, keepdims=True)
        topi_s, w_s = _run_bucket_topk(
            pb0, nch, q2_s, phones_s.reshape(T, 1), p2_s.reshape(1, N),
            code_arr.reshape(1, N), h_s, p_sorted)
        qinv = jnp.zeros(T, jnp.int32).at[qperm].set(
            jnp.arange(T, dtype=jnp.int32))
        topi_o = topi_s[qinv]
        w_o = w_s[qinv]
        g = _make_sc_gather(N, D, T * K)(p_sorted, topi_o.reshape(T * K))
        return _run_combine(g, w_o)

    def dense_path():
        q2 = jnp.sum(h_clean * h_clean, axis=1, keepdims=True)
        p2_flat = jnp.sum(prototypes * prototypes, axis=1)
        code2d = jnp.where(code_v == 64, jnp.int32(-1), code_v).reshape(1, N)
        topi, w = _run_dense_topk(q2, phones_i.reshape(T, 1),
                                  p2_flat.reshape(1, N), code2d, h_clean,
                                  prototypes)
        g = _make_sc_gather(N, D, T * K)(prototypes, topi.reshape(T * K))
        return _run_combine(g, w)

    return lax.cond(fast_ok, fast_path, dense_path)


# speedup vs baseline: 1.0298x; 1.0298x over previous
import functools

import jax
import jax.numpy as jnp
from jax import lax
from jax.experimental import pallas as pl
from jax.experimental.pallas import tpu as pltpu
from jax.experimental.pallas import tpu_sc as plsc

K = 8
BIG = 1e9
N_BLK = 512
LANES = 128
T_TILE = 256
QT = 16
NCH = 12
BCAP = 1024


def _insert_stream(md, base_col, vals, inds, rows):
    cur_v = [vals[s] for s in range(K)]
    cur_i = [inds[s] for s in range(K)]
    for g in range(N_BLK // LANES):
        v = md[:, g * LANES:(g + 1) * LANES]
        vidx = (base_col + g * LANES
                + lax.broadcasted_iota(jnp.int32, (rows, LANES), 1))
        c = [v < cur_v[s] for s in range(K)]
        new_v, new_i = [], []
        for s in range(K):
            if s == 0:
                new_v.append(jnp.where(c[0], v, cur_v[0]))
                new_i.append(jnp.where(c[0], vidx, cur_i[0]))
            else:
                new_v.append(jnp.where(c[s], jnp.where(c[s - 1], cur_v[s - 1],
                                                       v), cur_v[s]))
                new_i.append(jnp.where(c[s], jnp.where(c[s - 1], cur_i[s - 1],
                                                       vidx), cur_i[s]))
        cur_v, cur_i = new_v, new_i
    for s in range(K):
        vals[s] = cur_v[s]
        inds[s] = cur_i[s]


def _masked_dist(q2, p2, code, ph, hb, pb):
    cross = lax.dot_general(
        hb.astype(jnp.bfloat16), pb.astype(jnp.bfloat16),
        dimension_numbers=(((1,), (1,)), ((), ())),
        preferred_element_type=jnp.float32,
    )
    d2 = (q2 + p2) - 2.0 * cross
    dist = jnp.sqrt(jnp.maximum(d2, 1e-12))
    return jnp.where(code == ph, dist, jnp.float32(BIG))


def _merge_and_softmax(vals, inds, idx_out_ref, w_out_ref):
    cv = jnp.stack([vals[s] for s in range(K)])
    ci = jnp.stack([inds[s] for s in range(K)])
    outd, outi = [], []
    for _ in range(K):
        m = jnp.min(jnp.min(cv, axis=0), axis=1, keepdims=True)
        eq = cv == m[None, :, :]
        imin = jnp.min(jnp.min(jnp.where(eq, ci, jnp.int32(2**30)), axis=0),
                       axis=1, keepdims=True)
        outd.append(m)
        outi.append(imin)
        kill = eq & (ci == imin[None, :, :])
        cv = jnp.where(kill, jnp.inf, cv)
    topd = jnp.concatenate(outd, axis=1)
    topi = jnp.concatenate(outi, axis=1)
    unnorm = jnp.exp(-(topd - topd[:, 0:1]))
    w = unnorm / jnp.sum(unnorm, axis=1, keepdims=True)
    idx_out_ref[...] = topi
    w_out_ref[...] = w


def _dense_body(q2_ref, ph_ref, p2_ref, code_ref, h_ref, pr_ref,
                idx_out_ref, w_out_ref, vals, inds, *, n_blocks):
    j = pl.program_id(1)

    @pl.when(j == 0)
    def _init():
        vals[...] = jnp.full((K, T_TILE, LANES), jnp.inf, jnp.float32)
        inds[...] = jnp.zeros((K, T_TILE, LANES), jnp.int32)

    md = _masked_dist(q2_ref[...], p2_ref[...], code_ref[...], ph_ref[...],
                      h_ref[...], pr_ref[...])
    _insert_stream(md, j * N_BLK, vals, inds, T_TILE)

    @pl.when(j == n_blocks - 1)
    def _merge():
        _merge_and_softmax(vals, inds, idx_out_ref, w_out_ref)


def _bucket_body(pb0_ref, nch_ref, q2_ref, ph_ref, p2_ref, code_ref, h_ref,
                 pr_ref, idx_out_ref, w_out_ref, vals, inds):
    i = pl.program_id(0)
    c = pl.program_id(1)

    @pl.when(c == 0)
    def _init():
        vals[...] = jnp.full((K, QT, LANES), jnp.inf, jnp.float32)
        inds[...] = jnp.zeros((K, QT, LANES), jnp.int32)

    @pl.when(c < nch_ref[i])
    def _insert():
        blk = pb0_ref[i] + jnp.minimum(c, nch_ref[i] - 1)
        md = _masked_dist(q2_ref[...], p2_ref[...], code_ref[...], ph_ref[...],
                          h_ref[...], pr_ref[...])
        _insert_stream(md, blk * N_BLK, vals, inds, QT)

    @pl.when(c == NCH - 1)
    def _merge():
        _merge_and_softmax(vals, inds, idx_out_ref, w_out_ref)


def _fast_blk(i, c, pb0, nch):
    return pb0[i] + jnp.minimum(c, nch[i] - 1)


def _run_bucket_topk(pb0, nch, q2_s, phones_s2d, p2_s2d, code_s2d, h_s,
                     p_sorted):
    T, D = h_s.shape
    grid_spec = pltpu.PrefetchScalarGridSpec(
        num_scalar_prefetch=2,
        grid=(T // QT, NCH),
        in_specs=[
            pl.BlockSpec((QT, 1), lambda i, c, pb0, nch: (i, 0)),
            pl.BlockSpec((QT, 1), lambda i, c, pb0, nch: (i, 0)),
            pl.BlockSpec((1, N_BLK),
                         lambda i, c, pb0, nch: (0, _fast_blk(i, c, pb0, nch))),
            pl.BlockSpec((1, N_BLK),
                         lambda i, c, pb0, nch: (0, _fast_blk(i, c, pb0, nch))),
            pl.BlockSpec((QT, D), lambda i, c, pb0, nch: (i, 0)),
            pl.BlockSpec((N_BLK, D),
                         lambda i, c, pb0, nch: (_fast_blk(i, c, pb0, nch), 0)),
        ],
        out_specs=[
            pl.BlockSpec((QT, K), lambda i, c, pb0, nch: (i, 0)),
            pl.BlockSpec((QT, K), lambda i, c, pb0, nch: (i, 0)),
        ],
        scratch_shapes=[
            pltpu.VMEM((K, QT, LANES), jnp.float32),
            pltpu.VMEM((K, QT, LANES), jnp.int32),
        ],
    )
    return pl.pallas_call(
        _bucket_body,
        grid_spec=grid_spec,
        out_shape=[
            jax.ShapeDtypeStruct((T, K), jnp.int32),
            jax.ShapeDtypeStruct((T, K), jnp.float32),
        ],
        compiler_params=pltpu.CompilerParams(
            dimension_semantics=("arbitrary", "arbitrary"),
        ),
    )(pb0, nch, q2_s, phones_s2d, p2_s2d, code_s2d, h_s, p_sorted)


def _run_dense_topk(q2, phones2d, p2, code2d, h_clean, prototypes):
    T, D = h_clean.shape
    N = prototypes.shape[0]
    n_t, n_blocks = T // T_TILE, N // N_BLK
    kern = functools.partial(_dense_body, n_blocks=n_blocks)
    return pl.pallas_call(
        kern,
        grid=(n_t, n_blocks),
        in_specs=[
            pl.BlockSpec((T_TILE, 1), lambda i, j: (i, 0)),
            pl.BlockSpec((T_TILE, 1), lambda i, j: (i, 0)),
            pl.BlockSpec((1, N_BLK), lambda i, j: (0, j)),
            pl.BlockSpec((1, N_BLK), lambda i, j: (0, j)),
            pl.BlockSpec((T_TILE, D), lambda i, j: (i, 0)),
            pl.BlockSpec((N_BLK, D), lambda i, j: (j, 0)),
        ],
        out_specs=[
            pl.BlockSpec((T_TILE, K), lambda i, j: (i, 0)),
            pl.BlockSpec((T_TILE, K), lambda i, j: (i, 0)),
        ],
        out_shape=[
            jax.ShapeDtypeStruct((T, K), jnp.int32),
            jax.ShapeDtypeStruct((T, K), jnp.float32),
        ],
        scratch_shapes=[
            pltpu.VMEM((K, T_TILE, LANES), jnp.float32),
            pltpu.VMEM((K, T_TILE, LANES), jnp.int32),
        ],
        compiler_params=pltpu.CompilerParams(
            dimension_semantics=("arbitrary", "arbitrary"),
        ),
    )(q2, phones2d, p2, code2d, h_clean, prototypes)


def _make_sc_gather(V, D, B, chunk=None):
    nw = 32
    b_per_w = B // nw
    chunk = chunk or b_per_w
    n_ch = b_per_w // chunk
    mesh = plsc.VectorSubcoreMesh(core_axis_name="c", subcore_axis_name="s")

    @functools.partial(
        pl.kernel, mesh=mesh,
        out_type=jax.ShapeDtypeStruct((B, D), jnp.float32),
        scratch_types=[
            pltpu.VMEM((chunk,), jnp.int32),
            pltpu.VMEM((chunk, D), jnp.float32),
            pltpu.SemaphoreType.DMA,
        ],
    )
    def sc_gather(table_hbm, idx_hbm, out_hbm, idx_v, rows_v, sem):
        wid = lax.axis_index("s") * 2 + lax.axis_index("c")
        for ch in range(n_ch):
            base = wid * b_per_w + ch * chunk
            pltpu.sync_copy(idx_hbm.at[pl.ds(base, chunk)], idx_v)
            pltpu.async_copy(table_hbm.at[idx_v], rows_v, sem).wait()
            pltpu.sync_copy(rows_v, out_hbm.at[pl.ds(base, chunk)])

    return sc_gather


def _make_sc_gather_scatter(D, B, chunk):
    nw = 32
    b_per_w = B // nw
    n_ch = b_per_w // chunk
    mesh = plsc.VectorSubcoreMesh(core_axis_name="c", subcore_axis_name="s")

    @functools.partial(
        pl.kernel, mesh=mesh,
        out_type=jax.ShapeDtypeStruct((B, D), jnp.float32),
        scratch_types=[
            pltpu.VMEM((chunk,), jnp.int32),
            pltpu.VMEM((chunk,), jnp.int32),
            pltpu.VMEM((chunk, D), jnp.float32),
            pltpu.SemaphoreType.DMA,
        ],
    )
    def sc_gs(table_hbm, gidx_hbm, dest_hbm, out_hbm, idx_v, dest_v, rows_v,
              sem):
        wid = lax.axis_index("s") * 2 + lax.axis_index("c")
        for ch in range(n_ch):
            base = wid * b_per_w + ch * chunk
            pltpu.sync_copy(gidx_hbm.at[pl.ds(base, chunk)], idx_v)
            pltpu.sync_copy(dest_hbm.at[pl.ds(base, chunk)], dest_v)
            pltpu.async_copy(table_hbm.at[idx_v], rows_v, sem).wait()
            pltpu.async_copy(rows_v, out_hbm.at[dest_v], sem).wait()

    return sc_gs


def _bitrev16(x):
    x = ((x & 0x00FF) << 8) | ((x >> 8) & 0x00FF)
    x = ((x & 0x0F0F) << 4) | ((x >> 4) & 0x0F0F)
    x = ((x & 0x3333) << 2) | ((x >> 2) & 0x3333)
    x = ((x & 0x5555) << 1) | ((x >> 1) & 0x5555)
    return x


def _combine_body(g_ref, w_ref, out_ref):
    g = g_ref[...]
    w = w_ref[...]
    out_ref[...] = jnp.sum(w[:, :, None] * g, axis=1)


def _run_combine(gathered, w):
    T = w.shape[0]
    g3 = gathered.reshape(T, K, 256)
    return pl.pallas_call(
        _combine_body,
        grid=(T // T_TILE,),
        in_specs=[
            pl.BlockSpec((T_TILE, K, 256), lambda i: (i, 0, 0)),
            pl.BlockSpec((T_TILE, K), lambda i: (i, 0)),
        ],
        out_specs=pl.BlockSpec((T_TILE, 256), lambda i: (i, 0)),
        out_shape=jax.ShapeDtypeStruct((T, 256), jnp.float32),
    )(g3, w)


def kernel(h_clean, phones, target_gender, prototypes, proto_phones,
           proto_genders):
    T, D = h_clean.shape
    N = prototypes.shape[0]
    phones_i = phones.astype(jnp.int32)

    code_v = jnp.where(proto_genders == target_gender, proto_phones,
                       jnp.int32(64)).astype(jnp.int32)
    code_blk = code_v.reshape(N // LANES, LANES)
    oh3 = (code_blk[:, :, None]
           == jnp.arange(64, dtype=jnp.int32)[None, None, :]).astype(jnp.int32)
    blk_cnt = oh3.sum(axis=1)
    counts64 = blk_cnt.sum(axis=0)

    qperm = jnp.argsort(phones_i, stable=True).astype(jnp.int32)
    phones_s = jnp.sort(phones_i)
    pfirst = phones_s[0::QT]
    plast = phones_s[QT - 1::QT]
    nch = (2 * (plast - pfirst + 1)).astype(jnp.int32)
    pb0 = (2 * pfirst).astype(jnp.int32)

    fast_ok = (jnp.max(counts64) <= BCAP) & (jnp.max(nch) <= NCH)

    def fast_path():
        blk_base = jnp.cumsum(blk_cnt, axis=0) - blk_cnt
        within = jnp.cumsum(oh3, axis=1) - oh3
        rank = (oh3 * (blk_base[:, None, :] + within)).sum(axis=2).reshape(N)
        valid = (code_v < 64) & (rank < BCAP)
        pos = jnp.where(valid, code_v * BCAP + rank, N)
        code_arr = jnp.full(N, -2, jnp.int32).at[pos].set(code_v, mode="drop")
        pos_sh = jnp.where(valid, _bitrev16(pos), N)
        gidx = jnp.zeros(N, jnp.int32).at[pos_sh].set(
            jnp.arange(N, dtype=jnp.int32), mode="drop")
        dest = _bitrev16(jnp.arange(N, dtype=jnp.int32))
        p_sorted = _make_sc_gather_scatter(D, N, 128)(prototypes, gidx, dest)
        p2_s = jnp.sum(p_sorted * p_sorted, axis=1)
        h_s = _make_sc_gather(T, D, T)(h_clean, qperm)
        q2_s = jnp.sum(h_s * h_s, axis=1, keepdims=True)
        topi_s, w_s = _run_bucket_topk(
            pb0, nch, q2_s, phones_s.reshape(T, 1), p2_s.reshape(1, N),
            code_arr.reshape(1, N), h_s, p_sorted)
        qinv = jnp.zeros(T, jnp.int32).at[qperm].set(
            jnp.arange(T, dtype=jnp.int32))
        topi_o = topi_s[qinv]
        w_o = w_s[qinv]
        g = _make_sc_gather(N, D, T * K)(p_sorted, topi_o.reshape(T * K))
        return _run_combine(g, w_o)

    def dense_path():
        q2 = jnp.sum(h_clean * h_clean, axis=1, keepdims=True)
        p2_flat = jnp.sum(prototypes * prototypes, axis=1)
        code2d = jnp.where(code_v == 64, jnp.int32(-1), code_v).reshape(1, N)
        topi, w = _run_dense_topk(q2, phones_i.reshape(T, 1),
                                  p2_flat.reshape(1, N), code2d, h_clean,
                                  prototypes)
        g = _make_sc_gather(N, D, T * K)(prototypes, topi.reshape(T * K))
        return _run_combine(g, w)

    return lax.cond(fast_ok, fast_path, dense_path)


# speedup vs baseline: 3.0233x; 2.9358x over previous
import functools

import jax
import jax.numpy as jnp
from jax import lax
from jax.experimental import pallas as pl
from jax.experimental.pallas import tpu as pltpu
from jax.experimental.pallas import tpu_sc as plsc

K = 8
BIG = 1e9
N_BLK = 512
LANES = 128
T_TILE = 256


def _insert_stream(md, base_col, vals, inds, rows):
    cur_v = [vals[s] for s in range(K)]
    cur_i = [inds[s] for s in range(K)]
    for g in range(N_BLK // LANES):
        v = md[:, g * LANES:(g + 1) * LANES]
        vidx = (base_col + g * LANES
                + lax.broadcasted_iota(jnp.int32, (rows, LANES), 1))
        c = [v < cur_v[s] for s in range(K)]
        new_v, new_i = [], []
        for s in range(K):
            if s == 0:
                new_v.append(jnp.where(c[0], v, cur_v[0]))
                new_i.append(jnp.where(c[0], vidx, cur_i[0]))
            else:
                new_v.append(jnp.where(c[s], jnp.where(c[s - 1], cur_v[s - 1],
                                                       v), cur_v[s]))
                new_i.append(jnp.where(c[s], jnp.where(c[s - 1], cur_i[s - 1],
                                                       vidx), cur_i[s]))
        cur_v, cur_i = new_v, new_i
    for s in range(K):
        vals[s] = cur_v[s]
        inds[s] = cur_i[s]


def _masked_dist(q2, p2, code, ph, hb, pb):
    cross = lax.dot_general(
        hb.astype(jnp.bfloat16), pb.astype(jnp.bfloat16),
        dimension_numbers=(((1,), (1,)), ((), ())),
        preferred_element_type=jnp.float32,
    )
    d2 = (q2 + p2) - 2.0 * cross
    dist = jnp.sqrt(jnp.maximum(d2, 1e-12))
    return jnp.where(code == ph, dist, jnp.float32(BIG))


def _merge_and_softmax(vals, inds, idx_out_ref, w_out_ref):
    cv = jnp.stack([vals[s] for s in range(K)])
    ci = jnp.stack([inds[s] for s in range(K)])
    outd, outi = [], []
    for _ in range(K):
        m = jnp.min(jnp.min(cv, axis=0), axis=1, keepdims=True)
        eq = cv == m[None, :, :]
        imin = jnp.min(jnp.min(jnp.where(eq, ci, jnp.int32(2**30)), axis=0),
                       axis=1, keepdims=True)
        outd.append(m)
        outi.append(imin)
        kill = eq & (ci == imin[None, :, :])
        cv = jnp.where(kill, jnp.inf, cv)
    topd = jnp.concatenate(outd, axis=1)
    topi = jnp.concatenate(outi, axis=1)
    unnorm = jnp.exp(-(topd - topd[:, 0:1]))
    w = unnorm / jnp.sum(unnorm, axis=1, keepdims=True)
    idx_out_ref[...] = topi
    w_out_ref[...] = w


def _dense_body(q2_ref, ph_ref, p2_ref, code_ref, h_ref, pr_ref,
                idx_out_ref, w_out_ref, vals, inds, *, n_blocks):
    j = pl.program_id(1)

    @pl.when(j == 0)
    def _init():
        vals[...] = jnp.full((K, T_TILE, LANES), jnp.inf, jnp.float32)
        inds[...] = jnp.zeros((K, T_TILE, LANES), jnp.int32)

    md = _masked_dist(q2_ref[...], p2_ref[...], code_ref[...], ph_ref[...],
                      h_ref[...], pr_ref[...])
    _insert_stream(md, j * N_BLK, vals, inds, T_TILE)

    @pl.when(j == n_blocks - 1)
    def _merge():
        _merge_and_softmax(vals, inds, idx_out_ref, w_out_ref)


def _run_dense_topk(q2, phones2d, p2, code2d, h_clean, prototypes):
    T, D = h_clean.shape
    N = prototypes.shape[0]
    n_t, n_blocks = T // T_TILE, N // N_BLK
    kern = functools.partial(_dense_body, n_blocks=n_blocks)
    return pl.pallas_call(
        kern,
        grid=(n_t, n_blocks),
        in_specs=[
            pl.BlockSpec((T_TILE, 1), lambda i, j: (i, 0)),
            pl.BlockSpec((T_TILE, 1), lambda i, j: (i, 0)),
            pl.BlockSpec((1, N_BLK), lambda i, j: (0, j)),
            pl.BlockSpec((1, N_BLK), lambda i, j: (0, j)),
            pl.BlockSpec((T_TILE, D), lambda i, j: (i, 0)),
            pl.BlockSpec((N_BLK, D), lambda i, j: (j, 0)),
        ],
        out_specs=[
            pl.BlockSpec((T_TILE, K), lambda i, j: (i, 0)),
            pl.BlockSpec((T_TILE, K), lambda i, j: (i, 0)),
        ],
        out_shape=[
            jax.ShapeDtypeStruct((T, K), jnp.int32),
            jax.ShapeDtypeStruct((T, K), jnp.float32),
        ],
        scratch_shapes=[
            pltpu.VMEM((K, T_TILE, LANES), jnp.float32),
            pltpu.VMEM((K, T_TILE, LANES), jnp.int32),
        ],
        compiler_params=pltpu.CompilerParams(
            dimension_semantics=("arbitrary", "arbitrary"),
        ),
    )(q2, phones2d, p2, code2d, h_clean, prototypes)


def _make_sc_gather(D, B):
    nw = 32
    b_per_w = B // nw
    mesh = plsc.VectorSubcoreMesh(core_axis_name="c", subcore_axis_name="s")

    @functools.partial(
        pl.kernel, mesh=mesh,
        out_type=jax.ShapeDtypeStruct((B, D), jnp.float32),
        scratch_types=[
            pltpu.VMEM((b_per_w,), jnp.int32),
            pltpu.VMEM((b_per_w, D), jnp.float32),
            pltpu.SemaphoreType.DMA,
        ],
    )
    def sc_gather(table_hbm, idx_hbm, out_hbm, idx_v, rows_v, sem):
        wid = lax.axis_index("s") * 2 + lax.axis_index("c")
        base = wid * b_per_w
        pltpu.sync_copy(idx_hbm.at[pl.ds(base, b_per_w)], idx_v)
        pltpu.async_copy(table_hbm.at[idx_v], rows_v, sem).wait()
        pltpu.sync_copy(rows_v, out_hbm.at[pl.ds(base, b_per_w)])

    return sc_gather


def _combine_body(g_ref, w_ref, out_ref):
    g = g_ref[...]
    w = w_ref[...]
    out_ref[...] = jnp.sum(w[:, :, None] * g, axis=1)


def _run_combine(gathered, w):
    T = w.shape[0]
    g3 = gathered.reshape(T, K, 256)
    return pl.pallas_call(
        _combine_body,
        grid=(T // T_TILE,),
        in_specs=[
            pl.BlockSpec((T_TILE, K, 256), lambda i: (i, 0, 0)),
            pl.BlockSpec((T_TILE, K), lambda i: (i, 0)),
        ],
        out_specs=pl.BlockSpec((T_TILE, 256), lambda i: (i, 0)),
        out_shape=jax.ShapeDtypeStruct((T, 256), jnp.float32),
    )(g3, w)


def kernel(h_clean, phones, target_gender, prototypes, proto_phones,
           proto_genders):
    T, D = h_clean.shape
    N = prototypes.shape[0]
    q2 = jnp.sum(h_clean * h_clean, axis=1, keepdims=True)
    p2 = jnp.sum(prototypes * prototypes, axis=1)
    code = jnp.where(proto_genders == target_gender, proto_phones,
                     jnp.int32(-1)).astype(jnp.int32)
    topi, w = _run_dense_topk(q2, phones.astype(jnp.int32).reshape(T, 1),
                              p2.reshape(1, N), code.reshape(1, N), h_clean,
                              prototypes)
    gathered = _make_sc_gather(D, T * K)(prototypes, topi.reshape(T * K))
    return _run_combine(gathered, w)
